# Initial kernel scaffold; baseline (speedup 1.0000x reference)
import functools

import jax
import jax.numpy as jnp
from jax import lax
from jax.experimental import pallas as pl
from jax.experimental.pallas import tpu as pltpu
from jax.experimental.pallas import tpu_sc as plsc

BINS = 1000000
D = 32
NC = 2
NS = 16
NW = NC * NS
B = 4096 * 200
PER_W = B // NW
CH = 1024
G = 128
NG = CH // G
NCHUNK = PER_W // CH


def _build(interpret=False):
  mesh = plsc.VectorSubcoreMesh(core_axis_name="c", subcore_axis_name="s")

  @functools.partial(
      pl.kernel,
      out_type=jax.ShapeDtypeStruct((B, D), jnp.float32),
      mesh=mesh,
      scratch_types=[
          pltpu.VMEM((CH,), jnp.float32),
          pltpu.VMEM((NG, G), jnp.int32),
          pltpu.VMEM((CH, D), jnp.float32),
          pltpu.SemaphoreType.DMA,
      ],
      interpret=interpret,
  )
  def table_lookup(x_hbm, table_hbm, out_hbm, x_v, idx_v, rows_v, sem):
    wid = lax.axis_index("s") * NC + lax.axis_index("c")
    base = wid * PER_W

    def chunk(c, carry):
      off = base + c * CH
      pltpu.sync_copy(x_hbm.at[pl.ds(off, CH)], x_v)
      for g in range(NG):
        for i in range(G // 16):
          xv = x_v[pl.ds(g * G + i * 16, 16)]
          ii = jnp.minimum((xv * float(BINS)).astype(jnp.int32), BINS - 1)
          idx_v[g, pl.ds(i * 16, 16)] = ii
      copies = [
          pltpu.make_async_copy(
              table_hbm.at[idx_v.at[g]],
              rows_v.at[pl.ds(g * G, G)],
              sem,
          )
          for g in range(NG)
      ]
      for cp in copies:
        cp.start()
      for cp in copies:
        cp.wait()
      pltpu.sync_copy(rows_v, out_hbm.at[pl.ds(off, CH)])
      return carry

    lax.fori_loop(0, NCHUNK, chunk, 0)

  return table_lookup


_lookup = _build()


def kernel(x, table):
  xf = x.reshape(B)
  out = _lookup(xf, table)
  return out.reshape(4096, 200, D)


# speedup vs baseline: 1.4697x; 1.4697x over previous
import functools

import jax
import jax.numpy as jnp
from jax import lax
from jax.experimental import pallas as pl
from jax.experimental.pallas import tpu as pltpu
from jax.experimental.pallas import tpu_sc as plsc

BINS = 1000000
D = 32
NC = 2
NS = 16
NW = NC * NS
B = 4096 * 200
PER_W = B // NW
CH = 1024
G = 128
NG = CH // G
NCHUNK = PER_W // CH


def _build(interpret=False):
  mesh = plsc.VectorSubcoreMesh(core_axis_name="c", subcore_axis_name="s")

  @functools.partial(
      pl.kernel,
      out_type=jax.ShapeDtypeStruct((B, D), jnp.float32),
      mesh=mesh,
      scratch_types=[
          pltpu.VMEM((CH,), jnp.float32),
          pltpu.VMEM((NG, G), jnp.int32),
          pltpu.VMEM((CH, D), jnp.float32),
          pltpu.SemaphoreType.DMA,
      ],
      compiler_params=pltpu.CompilerParams(use_tc_tiling_on_sc=False),
      interpret=interpret,
  )
  def table_lookup(x_hbm, table_hbm, out_hbm, x_v, idx_v, rows_v, sem):
    wid = lax.axis_index("s") * NC + lax.axis_index("c")
    base = wid * PER_W

    def chunk(c, carry):
      off = base + c * CH
      pltpu.sync_copy(x_hbm.at[pl.ds(off, CH)], x_v)
      for g in range(NG):
        for i in range(G // 16):
          xv = x_v[pl.ds(g * G + i * 16, 16)]
          ii = jnp.minimum((xv * float(BINS)).astype(jnp.int32), BINS - 1)
          idx_v[g, pl.ds(i * 16, 16)] = ii
      copies = [
          pltpu.make_async_copy(
              table_hbm.at[idx_v.at[g]],
              rows_v.at[pl.ds(g * G, G)],
              sem,
          )
          for g in range(NG)
      ]
      for cp in copies:
        cp.start()
      for cp in copies:
        cp.wait()
      pltpu.sync_copy(rows_v, out_hbm.at[pl.ds(off, CH)])
      return carry

    lax.fori_loop(0, NCHUNK, chunk, 0)

  return table_lookup


_lookup = _build()


def kernel(x, table):
  xf = x.reshape(B)
  out = _lookup(xf, table)
  return out.reshape(4096, 200, D)


# speedup vs baseline: 1.4815x; 1.0080x over previous
import functools

import jax
import jax.numpy as jnp
from jax import lax
from jax.experimental import pallas as pl
from jax.experimental.pallas import tpu as pltpu
from jax.experimental.pallas import tpu_sc as plsc

BINS = 1000000
D = 32
NC = 2
NS = 16
NW = NC * NS
B = 4096 * 200
PER_W = B // NW
CH = 512
G = 128
NG = CH // G
NCHUNK = PER_W // CH
HALF = NCHUNK // 2
UNROLL = 16


def _build(interpret=False):
  mesh = plsc.VectorSubcoreMesh(core_axis_name="c", subcore_axis_name="s")

  @functools.partial(
      pl.kernel,
      out_type=jax.ShapeDtypeStruct((B, D), jnp.float32),
      mesh=mesh,
      scratch_types=[
          pltpu.VMEM((PER_W,), jnp.float32),
          pltpu.VMEM((PER_W,), jnp.int32),
          pltpu.VMEM((CH, D), jnp.float32),
          pltpu.VMEM((CH, D), jnp.float32),
          pltpu.SemaphoreType.DMA,
          pltpu.SemaphoreType.DMA,
          pltpu.SemaphoreType.DMA,
      ],
      compiler_params=pltpu.CompilerParams(use_tc_tiling_on_sc=False),
      interpret=interpret,
  )
  def table_lookup(x_hbm, table_hbm, out_hbm,
                   x_all, idx_all, rows0, rows1, sem_g, sem_s0, sem_s1):
    wid = lax.axis_index("s") * NC + lax.axis_index("c")
    base = wid * PER_W

    pltpu.sync_copy(x_hbm.at[pl.ds(base, PER_W)], x_all)

    def bucketize(t, carry):
      for i in range(UNROLL):
        off = t * (16 * UNROLL) + i * 16
        xv = x_all[pl.ds(off, 16)]
        idx_all[pl.ds(off, 16)] = jnp.minimum(
            (xv * float(BINS)).astype(jnp.int32), BINS - 1)
      return carry

    lax.fori_loop(0, PER_W // (16 * UNROLL), bucketize, 0)

    pltpu.make_async_copy(
        rows0, out_hbm.at[pl.ds(base + (NCHUNK - 2) * CH, CH)], sem_s0).start()
    pltpu.make_async_copy(
        rows1, out_hbm.at[pl.ds(base + (NCHUNK - 1) * CH, CH)], sem_s1).start()

    def pipe(t, carry):
      for b in range(2):
        rows = rows0 if b == 0 else rows1
        sem_s = sem_s0 if b == 0 else sem_s1
        c = 2 * t + b
        off = base + c * CH
        pltpu.make_async_copy(rows, out_hbm.at[pl.ds(off, CH)], sem_s).wait()
        gs = [
            pltpu.make_async_copy(
                table_hbm.at[idx_all.at[pl.ds(c * CH + g * G, G)]],
                rows.at[pl.ds(g * G, G)],
                sem_g,
            )
            for g in range(NG)
        ]
        for cp in gs:
          cp.start()
        for cp in gs:
          cp.wait()
        pltpu.make_async_copy(rows, out_hbm.at[pl.ds(off, CH)], sem_s).start()
      return carry

    lax.fori_loop(0, HALF, pipe, 0)

    pltpu.make_async_copy(rows0, out_hbm.at[pl.ds(base, CH)], sem_s0).wait()
    pltpu.make_async_copy(rows1, out_hbm.at[pl.ds(base, CH)], sem_s1).wait()

  return table_lookup


_lookup = _build()


def kernel(x, table):
  xf = x.reshape(B)
  out = _lookup(xf, table)
  return out.reshape(4096, 200, D)
